# baseline (speedup 1.0000x reference)
import functools
import math

import jax
import jax.numpy as jnp
from jax.experimental import pallas as pl

FLOW_LAYERS = 3
FLOW_HID = 16
TOK_BLOCK = 8192
SUB = 8


def _block_kernel(enc_ref, tv_ref, mw_ref, w0t_ref, b0_ref, w1t_ref, b1_ref,
                  w2t_ref, b2_ref, out_ref):
    j = pl.program_id(1)

    enc = enc_ref[0, 0].astype(jnp.bfloat16)
    h1t = jax.lax.dot_general(
        w0t_ref[...], enc, (((1,), (2,)), ((), ())),
        preferred_element_type=jnp.float32)
    h1t = jax.nn.relu(h1t + b0_ref[...]).astype(jnp.bfloat16)
    h2t = jax.lax.dot_general(
        w1t_ref[...], h1t, (((1,), (0,)), ((), ())),
        preferred_element_type=jnp.float32)
    h2t = jax.nn.relu(h2t + b1_ref[...]).astype(jnp.bfloat16)
    pt = jax.lax.dot_general(
        w2t_ref[...], h2t, (((1,), (0,)), ((), ())),
        preferred_element_type=jnp.float32)
    pt = pt + b2_ref[...]

    x = tv_ref[0, 0]
    logdet = jnp.zeros(x.shape, dtype=jnp.float32)
    delta = 1e-6
    for l in range(FLOW_LAYERS):
        base = l * 3 * FLOW_HID
        ap = pt[base:base + FLOW_HID]
        bp = pt[base + FLOW_HID:base + 2 * FLOW_HID]
        wp = pt[base + 2 * FLOW_HID:base + 3 * FLOW_HID]
        e1 = jnp.exp(-jnp.abs(ap))
        a = jnp.maximum(ap, 0.0) + jnp.log(1.0 + e1)
        ew = jnp.exp(wp)
        sew = jnp.sum(ew, axis=0)
        pre = a * x + bp
        e2 = jnp.exp(-jnp.abs(pre))
        r = 1.0 / (1.0 + e2)
        e2r = e2 * r
        ge = pre >= 0.0
        sig = jnp.where(ge, r, e2r)
        sigc = jnp.where(ge, e2r, r)
        ews = ew * sig
        x_pre = jnp.sum(ews, axis=0) / sew
        j_lin = jnp.sum(ews * (a * sigc), axis=0)
        logj = jnp.log(jnp.maximum(j_lin, 1e-37)) - jnp.log(sew)
        logdet = logdet + logj
        if l < FLOW_LAYERS - 1:
            xc = jnp.clip(x_pre, delta, 1.0 - delta)
            lxc = jnp.log(xc)
            l1m = jnp.log1p(-xc)
            x = lxc - l1m
            logdet = logdet - lxc - l1m

    partial = jnp.sum(mw_ref[0, 0] * logdet)

    @pl.when(j == 0)
    def _():
        out_ref[...] = jnp.zeros_like(out_ref)

    out_ref[...] = out_ref[...] - partial


def kernel(encoded, mask, true_value, W0, b0, W1, b1, W2, b2):
    B, S, T, D = encoded.shape
    N = S * T
    TB = TOK_BLOCK
    NT = N // TB
    TBL = TB // SUB

    enc5 = encoded.reshape(B, NT, SUB, TBL, D)
    tv4 = true_value.reshape(B, NT, SUB, TBL)
    m0 = mask.reshape(B, N)[0]
    mw = (~m0).astype(jnp.float32).reshape(1, NT, SUB, TBL)
    P = FLOW_LAYERS * 3 * FLOW_HID

    out = pl.pallas_call(
        _block_kernel,
        grid=(B, NT),
        in_specs=[
            pl.BlockSpec((1, 1, SUB, TBL, D), lambda b, j: (b, j, 0, 0, 0)),
            pl.BlockSpec((1, 1, SUB, TBL), lambda b, j: (b, j, 0, 0)),
            pl.BlockSpec((1, 1, SUB, TBL), lambda b, j: (0, j, 0, 0)),
            pl.BlockSpec((128, D), lambda b, j: (0, 0)),
            pl.BlockSpec((128, 1, 1), lambda b, j: (0, 0, 0)),
            pl.BlockSpec((128, 128), lambda b, j: (0, 0)),
            pl.BlockSpec((128, 1, 1), lambda b, j: (0, 0, 0)),
            pl.BlockSpec((P, 128), lambda b, j: (0, 0)),
            pl.BlockSpec((P, 1, 1), lambda b, j: (0, 0, 0)),
        ],
        out_specs=pl.BlockSpec((1, 1, 128), lambda b, j: (b, 0, 0)),
        out_shape=jax.ShapeDtypeStruct((B, 1, 128), jnp.float32),
    )(enc5, tv4, mw, W0.T.astype(jnp.bfloat16), b0.reshape(-1, 1, 1),
      W1.T.astype(jnp.bfloat16), b1.reshape(-1, 1, 1),
      W2.T.astype(jnp.bfloat16), b2.reshape(-1, 1, 1))
    return out[:, 0, 0]


# speedup vs baseline: 1.5190x; 1.5190x over previous
import functools
import math

import jax
import jax.numpy as jnp
from jax.experimental import pallas as pl

FLOW_LAYERS = 3
FLOW_HID = 16
TOK_BLOCK = 8192


def _block_kernel(enc_ref, tv_ref, mw_ref, w0t_ref, b0_ref, w1t_ref, b1_ref,
                  w2t_ref, b2_ref, out_ref):
    j = pl.program_id(1)

    enc = enc_ref[0].astype(jnp.bfloat16)
    h1t = jax.lax.dot_general(
        w0t_ref[...], enc, (((1,), (1,)), ((), ())),
        preferred_element_type=jnp.float32)
    h1t = jax.nn.relu(h1t + b0_ref[...]).astype(jnp.bfloat16)
    h2t = jax.lax.dot_general(
        w1t_ref[...], h1t, (((1,), (0,)), ((), ())),
        preferred_element_type=jnp.float32)
    h2t = jax.nn.relu(h2t + b1_ref[...]).astype(jnp.bfloat16)
    pt = jax.lax.dot_general(
        w2t_ref[...], h2t, (((1,), (0,)), ((), ())),
        preferred_element_type=jnp.float32)
    pt = pt + b2_ref[...]

    x = tv_ref[0]
    logdet = jnp.zeros(x.shape, dtype=jnp.float32)
    delta = 1e-6
    for l in range(FLOW_LAYERS):
        base = l * 3 * FLOW_HID
        ap = pt[base:base + FLOW_HID]
        bp = pt[base + FLOW_HID:base + 2 * FLOW_HID]
        wp = pt[base + 2 * FLOW_HID:base + 3 * FLOW_HID]
        e1 = jnp.exp(-jnp.abs(ap))
        a = jnp.maximum(ap, 0.0) + jnp.log(1.0 + e1)
        ew = jnp.exp(wp)
        sew = jnp.sum(ew, axis=0, keepdims=True)
        pre = a * x + bp
        e2 = jnp.exp(-jnp.abs(pre))
        r = 1.0 / (1.0 + e2)
        e2r = e2 * r
        ge = pre >= 0.0
        sig = jnp.where(ge, r, e2r)
        sigc = jnp.where(ge, e2r, r)
        ews = ew * sig
        x_pre = jnp.sum(ews, axis=0, keepdims=True) / sew
        j_lin = jnp.sum(ews * (a * sigc), axis=0, keepdims=True)
        logj = jnp.log(jnp.maximum(j_lin, 1e-37)) - jnp.log(sew)
        logdet = logdet + logj
        if l < FLOW_LAYERS - 1:
            xc = jnp.clip(x_pre, delta, 1.0 - delta)
            lxc = jnp.log(xc)
            l1m = jnp.log1p(-xc)
            x = lxc - l1m
            logdet = logdet - lxc - l1m

    partial = jnp.sum(mw_ref[0] * logdet)

    @pl.when(j == 0)
    def _():
        out_ref[...] = jnp.zeros_like(out_ref)

    out_ref[...] = out_ref[...] - partial


def kernel(encoded, mask, true_value, W0, b0, W1, b1, W2, b2):
    B, S, T, D = encoded.shape
    N = S * T
    TB = TOK_BLOCK
    NT = N // TB

    enc3 = encoded.reshape(B, N, D)
    tv3 = true_value.reshape(B, 1, N)
    m0 = mask.reshape(B, N)[0]
    mw = (~m0).astype(jnp.float32).reshape(1, 1, N)
    P = FLOW_LAYERS * 3 * FLOW_HID

    out = pl.pallas_call(
        _block_kernel,
        grid=(B, NT),
        in_specs=[
            pl.BlockSpec((1, TB, D), lambda b, j: (b, j, 0)),
            pl.BlockSpec((1, 1, TB), lambda b, j: (b, 0, j)),
            pl.BlockSpec((1, 1, TB), lambda b, j: (0, 0, j)),
            pl.BlockSpec((128, D), lambda b, j: (0, 0)),
            pl.BlockSpec((128, 1), lambda b, j: (0, 0)),
            pl.BlockSpec((128, 128), lambda b, j: (0, 0)),
            pl.BlockSpec((128, 1), lambda b, j: (0, 0)),
            pl.BlockSpec((P, 128), lambda b, j: (0, 0)),
            pl.BlockSpec((P, 1), lambda b, j: (0, 0)),
        ],
        out_specs=pl.BlockSpec((1, 1, 128), lambda b, j: (b, 0, 0)),
        out_shape=jax.ShapeDtypeStruct((B, 1, 128), jnp.float32),
    )(enc3, tv3, mw, W0.T.astype(jnp.bfloat16), b0.reshape(-1, 1),
      W1.T.astype(jnp.bfloat16), b1.reshape(-1, 1),
      W2.T.astype(jnp.bfloat16), b2.reshape(-1, 1))
    return out[:, 0, 0]


# speedup vs baseline: 1.5267x; 1.0051x over previous
import functools
import math

import jax
import jax.numpy as jnp
from jax.experimental import pallas as pl

FLOW_LAYERS = 3
FLOW_HID = 16
TOK_BLOCK = 16384


def _block_kernel(enc_ref, tv_ref, mw_ref, w0t_ref, b0_ref, w1t_ref, b1_ref,
                  w2t_ref, b2_ref, out_ref):
    j = pl.program_id(1)

    enc = enc_ref[0].astype(jnp.bfloat16)
    h1t = jax.lax.dot_general(
        w0t_ref[...], enc, (((1,), (1,)), ((), ())),
        preferred_element_type=jnp.float32)
    h1t = jax.nn.relu(h1t + b0_ref[...]).astype(jnp.bfloat16)
    h2t = jax.lax.dot_general(
        w1t_ref[...], h1t, (((1,), (0,)), ((), ())),
        preferred_element_type=jnp.float32)
    h2t = jax.nn.relu(h2t + b1_ref[...]).astype(jnp.bfloat16)
    pt = jax.lax.dot_general(
        w2t_ref[...], h2t, (((1,), (0,)), ((), ())),
        preferred_element_type=jnp.float32)
    pt = pt + b2_ref[...]

    x = tv_ref[0]
    logdet = jnp.zeros(x.shape, dtype=jnp.float32)
    delta = 1e-6
    for l in range(FLOW_LAYERS):
        base = l * 3 * FLOW_HID
        ap = pt[base:base + FLOW_HID]
        bp = pt[base + FLOW_HID:base + 2 * FLOW_HID]
        wp = pt[base + 2 * FLOW_HID:base + 3 * FLOW_HID]
        e1 = jnp.exp(-jnp.abs(ap))
        a = jnp.maximum(ap, 0.0) + jnp.log(1.0 + e1)
        ew = jnp.exp(wp)
        sew = jnp.sum(ew, axis=0, keepdims=True)
        pre = a * x + bp
        e2 = jnp.exp(-jnp.abs(pre))
        r = 1.0 / (1.0 + e2)
        e2r = e2 * r
        ge = pre >= 0.0
        sig = jnp.where(ge, r, e2r)
        sigc = jnp.where(ge, e2r, r)
        ews = ew * sig
        x_pre = jnp.sum(ews, axis=0, keepdims=True) / sew
        j_lin = jnp.sum(ews * (a * sigc), axis=0, keepdims=True)
        logj = jnp.log(jnp.maximum(j_lin, 1e-37)) - jnp.log(sew)
        logdet = logdet + logj
        if l < FLOW_LAYERS - 1:
            xc = jnp.clip(x_pre, delta, 1.0 - delta)
            lxc = jnp.log(xc)
            l1m = jnp.log1p(-xc)
            x = lxc - l1m
            logdet = logdet - lxc - l1m

    partial = jnp.sum(mw_ref[0] * logdet)

    @pl.when(j == 0)
    def _():
        out_ref[...] = jnp.zeros_like(out_ref)

    out_ref[...] = out_ref[...] - partial


def kernel(encoded, mask, true_value, W0, b0, W1, b1, W2, b2):
    B, S, T, D = encoded.shape
    N = S * T
    TB = TOK_BLOCK
    NT = N // TB

    enc3 = encoded.reshape(B, N, D)
    tv3 = true_value.reshape(B, 1, N)
    m0 = mask.reshape(B, N)[0]
    mw = (~m0).astype(jnp.float32).reshape(1, 1, N)
    P = FLOW_LAYERS * 3 * FLOW_HID

    out = pl.pallas_call(
        _block_kernel,
        grid=(B, NT),
        in_specs=[
            pl.BlockSpec((1, TB, D), lambda b, j: (b, j, 0)),
            pl.BlockSpec((1, 1, TB), lambda b, j: (b, 0, j)),
            pl.BlockSpec((1, 1, TB), lambda b, j: (0, 0, j)),
            pl.BlockSpec((128, D), lambda b, j: (0, 0)),
            pl.BlockSpec((128, 1), lambda b, j: (0, 0)),
            pl.BlockSpec((128, 128), lambda b, j: (0, 0)),
            pl.BlockSpec((128, 1), lambda b, j: (0, 0)),
            pl.BlockSpec((P, 128), lambda b, j: (0, 0)),
            pl.BlockSpec((P, 1), lambda b, j: (0, 0)),
        ],
        out_specs=pl.BlockSpec((1, 1, 128), lambda b, j: (b, 0, 0)),
        out_shape=jax.ShapeDtypeStruct((B, 1, 128), jnp.float32),
    )(enc3, tv3, mw, W0.T.astype(jnp.bfloat16), b0.reshape(-1, 1),
      W1.T.astype(jnp.bfloat16), b1.reshape(-1, 1),
      W2.T.astype(jnp.bfloat16), b2.reshape(-1, 1))
    return out[:, 0, 0]
